# Initial kernel scaffold; baseline (speedup 1.0000x reference)
import jax
import jax.numpy as jnp
from jax.experimental import pallas as pl


def kernel(x, pos, batch, params):
    raise NotImplementedError("write your pallas kernel here")


# speedup vs baseline: 1.1606x; 1.1606x over previous
import functools

import jax
import jax.numpy as jnp
from jax.experimental import pallas as pl

_B = 4
_EPS = 1e-5


def _round_up(v, m):
    return (v + m - 1) // m * m


def _fps_kernel(n, m, nb, p_ref, out_ref):
    npad = p_ref.shape[1]
    mpad = out_ref.shape[1]
    pt = p_ref[...]
    x0 = pt[0:nb, :]
    x1 = pt[nb:2 * nb, :]
    x2 = pt[2 * nb:3 * nb, :]
    col = jax.lax.broadcasted_iota(jnp.int32, (nb, npad), 1)
    mcol = jax.lax.broadcasted_iota(jnp.int32, (nb, mpad), 1)
    valid = col < n

    def body(i, carry):
        dists, last, idxs = carry
        sel = col == last
        p0 = jnp.sum(jnp.where(sel, x0, 0.0), axis=1, keepdims=True)
        p1 = jnp.sum(jnp.where(sel, x1, 0.0), axis=1, keepdims=True)
        p2 = jnp.sum(jnp.where(sel, x2, 0.0), axis=1, keepdims=True)
        d = (x0 - p0) ** 2 + (x1 - p1) ** 2 + (x2 - p2) ** 2
        dists = jnp.minimum(dists, jnp.where(valid, d, -jnp.inf))
        mx = jnp.max(dists, axis=1, keepdims=True)
        nxt = jnp.min(jnp.where(dists == mx, col, npad), axis=1,
                      keepdims=True).astype(jnp.int32)
        idxs = jnp.where(mcol == i + 1, nxt, idxs)
        return dists, nxt, idxs

    dists0 = jnp.where(valid, jnp.inf, -jnp.inf)
    last0 = jnp.zeros((nb, 1), jnp.int32)
    idxs0 = jnp.zeros((nb, mpad), jnp.int32)
    _, _, idxs = jax.lax.fori_loop(0, m - 1, body, (dists0, last0, idxs0))
    out_ref[...] = idxs


def _fps_pallas(pos_b, m):
    nb, n, _ = pos_b.shape
    npad = _round_up(n, 128)
    mpad = _round_up(m, 128)
    p = jnp.moveaxis(pos_b, 2, 0).reshape(3 * nb, n)
    p = jnp.pad(p, ((0, 0), (0, npad - n)))
    out = pl.pallas_call(
        functools.partial(_fps_kernel, n, m, nb),
        out_shape=jax.ShapeDtypeStruct((nb, mpad), jnp.int32),
    )(p)
    return out[:, :m]


def _mlp_kernel(nlayers, acts, x_ref, *refs):
    out_ref = refs[-1]
    h = x_ref[...]
    for i in range(nlayers):
        w = refs[2 * i][...]
        b = refs[2 * i + 1][...]
        h = jnp.dot(h, w, preferred_element_type=jnp.float32) + b
        a = acts[i]
        if a == 'sigmoid':
            h = jax.nn.sigmoid(h)
        elif a == 'relu':
            h = jax.nn.relu(h)
    out_ref[...] = h


def _mlp_pallas(x, ws, bs, acts, tile=256):
    rows, c0 = x.shape
    nl = len(ws)
    dims = [c0] + [w.shape[1] for w in ws]
    pdims = [_round_up(c, 128) for c in dims]
    rp = _round_up(rows, tile)
    xp = jnp.zeros((rp, pdims[0]), jnp.float32).at[:rows, :c0].set(x)
    ops = [xp]
    in_specs = [pl.BlockSpec((tile, pdims[0]), lambda i: (i, 0))]
    for li in range(nl):
        wp = jnp.zeros((pdims[li], pdims[li + 1]), jnp.float32)
        wp = wp.at[:dims[li], :dims[li + 1]].set(ws[li])
        bp = jnp.zeros((1, pdims[li + 1]), jnp.float32)
        bp = bp.at[:, :dims[li + 1]].set(bs[li])
        ops += [wp, bp]
        in_specs.append(pl.BlockSpec(wp.shape, lambda i: (0, 0)))
        in_specs.append(pl.BlockSpec(bp.shape, lambda i: (0, 0)))
    out = pl.pallas_call(
        functools.partial(_mlp_kernel, nl, tuple(acts)),
        grid=(rp // tile,),
        in_specs=in_specs,
        out_specs=pl.BlockSpec((tile, pdims[-1]), lambda i: (i, 0)),
        out_shape=jax.ShapeDtypeStruct((rp, pdims[-1]), jnp.float32),
    )(*ops)
    return out[:rows, :dims[-1]]


def _fold_chain(layers, hidden_act, last_act='none'):
    ws, bs, acts = [], [], []
    nl = len(layers)
    inv = 1.0 / jnp.sqrt(1.0 + _EPS)
    for i, p in enumerate(layers):
        if 'g' in p:
            sc = inv * p['g']
            ws.append(p['W'] * sc[None, :])
            bs.append(p['b'] * sc + p['bt'])
        else:
            ws.append(p['W'])
            bs.append(p['b'])
        acts.append(hidden_act if i < nl - 1 else last_act)
    return ws, bs, acts


def _radius(pos_x, pos_y, r, max_n):
    d2 = jnp.sum((pos_y[:, None, :] - pos_x[None, :, :]) ** 2, -1)
    n = pos_x.shape[0]
    score = jnp.where(d2 <= r * r, jnp.arange(n, dtype=jnp.int32)[None, :], n)
    neg, _ = jax.lax.top_k(-score, max_n)
    sc = -neg
    valid = sc < n
    return jnp.where(valid, sc, 0), valid


def _sa(xg, posg, r, m, ws, bs, acts):
    nb, n, dch = xg.shape
    idx = _fps_pallas(posg, m)
    posy = jnp.take_along_axis(posg, idx[..., None], axis=1)
    nbr, valid = jax.vmap(lambda px, py: _radius(px, py, r, 32))(posg, posy)
    nf = nbr.reshape(nb, m * 32, 1)
    fx = jnp.take_along_axis(xg, nf, axis=1).reshape(nb, m, 32, dch)
    fp = jnp.take_along_axis(posg, nf, axis=1).reshape(nb, m, 32, 3)
    fp = fp - posy[:, :, None, :]
    feat = jnp.concatenate([fx, fp], -1)
    msg = _mlp_pallas(feat.reshape(-1, dch + 3), ws, bs, acts)
    msg = msg.reshape(nb, m, 32, -1)
    msg = jnp.where(valid[..., None], msg, -jnp.inf)
    out = jnp.max(msg, axis=2)
    out = jnp.where(jnp.any(valid, axis=2)[..., None], out, 0.0)
    return jax.nn.relu(out), posy


def _up(xl, posl, xf, posf, cw, cb, ca, uw, ub, ua):
    nb, m, cl = xl.shape
    nf = posf.shape[1]
    d2 = jnp.sum((posl[:, :, None, :] - posf[:, None, :, :]) ** 2, -1)
    _, nbr = jax.lax.top_k(-d2, 64)
    pn = jnp.take_along_axis(posf, nbr.reshape(nb, m * 64, 1), axis=1)
    pn = pn.reshape(nb, m, 64, 3)
    feat = jnp.concatenate(
        [jnp.broadcast_to(xl[:, :, None, :], (nb, m, 64, cl)),
         posl[:, :, None, :] - pn], -1)
    msg = _mlp_pallas(feat.reshape(-1, cl + 3), cw, cb, ca)
    f = msg.shape[-1]
    msg = msg.reshape(nb, m, 64, f)
    feat_s = jnp.concatenate([xl, posl - posf[:, :m]], -1)
    msg_s = _mlp_pallas(feat_s.reshape(-1, cl + 3), cw, cb, ca)
    msg_s = msg_s.reshape(nb, m, f)
    out0 = jnp.full((nf, f), -jnp.inf, jnp.float32)

    def scat(nbi, ms, mss):
        o = out0.at[nbi.reshape(-1)].max(ms.reshape(-1, f))
        return o.at[jnp.arange(m)].max(mss)

    out = jax.vmap(scat)(nbr, msg, msg_s)
    out = jnp.where(jnp.isneginf(out), 0.0, out)
    xc = jnp.concatenate([out, xf], -1)
    h = _mlp_pallas(xc.reshape(-1, xc.shape[-1]), uw, ub, ua)
    return h.reshape(nb, nf, -1)


def kernel(x, pos, batch, params):
    n_total = pos.shape[0]
    n = n_total // _B
    xg = x.reshape(_B, n, -1)
    posg = pos.reshape(_B, n, 3)
    m1, m2 = n // 4, n // 16

    sa0 = _fold_chain(params['sa'][0], 'sigmoid')
    sa1 = _fold_chain(params['sa'][1], 'sigmoid')
    dec0 = _fold_chain(params['dec'][0], 'sigmoid')
    dec1 = _fold_chain(params['dec'][1], 'sigmoid')
    up0 = _fold_chain([params['up'][0]], 'relu', last_act='relu')
    up1 = _fold_chain([params['up'][1]], 'relu', last_act='relu')
    fc = _fold_chain(params['fc'], 'relu', last_act='sigmoid')

    x1, pos1 = _sa(xg, posg, 1.0, m1, *sa0)
    x2, pos2 = _sa(x1, pos1, 2.0, m2, *sa1)
    u1 = _up(x2, pos2, x1, pos1, *dec0, *up0)
    u0 = _up(u1, pos1, xg, posg, *dec1, *up1)
    out = _mlp_pallas(u0.reshape(n_total, -1), *fc)
    return (out, pos, batch)


# speedup vs baseline: 1.9698x; 1.6972x over previous
import functools

import jax
import jax.numpy as jnp
from jax.experimental import pallas as pl

_B = 4
_EPS = 1e-5


def _round_up(v, m):
    return (v + m - 1) // m * m


def _fps_kernel(n, m, nb, p_ref, out_ref):
    npad = p_ref.shape[1]
    mpad = out_ref.shape[1]
    pt = p_ref[...]
    x0 = pt[0:nb, :]
    x1 = pt[nb:2 * nb, :]
    x2 = pt[2 * nb:3 * nb, :]
    col = jax.lax.broadcasted_iota(jnp.int32, (nb, npad), 1)
    mcol = jax.lax.broadcasted_iota(jnp.int32, (nb, mpad), 1)
    valid = col < n

    def body(i, carry):
        dists, last, idxs = carry
        sel = col == last
        p0 = jnp.sum(jnp.where(sel, x0, 0.0), axis=1, keepdims=True)
        p1 = jnp.sum(jnp.where(sel, x1, 0.0), axis=1, keepdims=True)
        p2 = jnp.sum(jnp.where(sel, x2, 0.0), axis=1, keepdims=True)
        d = (x0 - p0) ** 2 + (x1 - p1) ** 2 + (x2 - p2) ** 2
        dists = jnp.minimum(dists, jnp.where(valid, d, -jnp.inf))
        mx = jnp.max(dists, axis=1, keepdims=True)
        nxt = jnp.min(jnp.where(dists == mx, col, npad), axis=1,
                      keepdims=True).astype(jnp.int32)
        idxs = jnp.where(mcol == i + 1, nxt, idxs)
        return dists, nxt, idxs

    dists0 = jnp.where(valid, jnp.inf, -jnp.inf)
    last0 = jnp.zeros((nb, 1), jnp.int32)
    idxs0 = jnp.zeros((nb, mpad), jnp.int32)
    _, _, idxs = jax.lax.fori_loop(0, m - 1, body, (dists0, last0, idxs0))
    out_ref[...] = idxs


def _fps_pallas(pos_b, m):
    nb, n, _ = pos_b.shape
    npad = _round_up(n, 128)
    mpad = _round_up(m, 128)
    p = jnp.moveaxis(pos_b, 2, 0).reshape(3 * nb, n)
    p = jnp.pad(p, ((0, 0), (0, npad - n)))
    out = pl.pallas_call(
        functools.partial(_fps_kernel, n, m, nb),
        out_shape=jax.ShapeDtypeStruct((nb, mpad), jnp.int32),
    )(p)
    return out[:, :m]


def _mlp_kernel(nlayers, acts, x_ref, *refs):
    out_ref = refs[-1]
    h = x_ref[...]
    for i in range(nlayers):
        w = refs[2 * i][...]
        b = refs[2 * i + 1][...]
        h = jnp.dot(h, w, preferred_element_type=jnp.float32) + b
        a = acts[i]
        if a == 'sigmoid':
            h = jax.nn.sigmoid(h)
        elif a == 'relu':
            h = jax.nn.relu(h)
    out_ref[...] = h


def _mlp_pallas(x, ws, bs, acts, tile=256):
    rows, c0 = x.shape
    nl = len(ws)
    dims = [c0] + [w.shape[1] for w in ws]
    pdims = [_round_up(c, 128) for c in dims]
    rp = _round_up(rows, tile)
    xp = jnp.zeros((rp, pdims[0]), jnp.float32).at[:rows, :c0].set(x)
    ops = [xp]
    in_specs = [pl.BlockSpec((tile, pdims[0]), lambda i: (i, 0))]
    for li in range(nl):
        wp = jnp.zeros((pdims[li], pdims[li + 1]), jnp.float32)
        wp = wp.at[:dims[li], :dims[li + 1]].set(ws[li])
        bp = jnp.zeros((1, pdims[li + 1]), jnp.float32)
        bp = bp.at[:, :dims[li + 1]].set(bs[li])
        ops += [wp, bp]
        in_specs.append(pl.BlockSpec(wp.shape, lambda i: (0, 0)))
        in_specs.append(pl.BlockSpec(bp.shape, lambda i: (0, 0)))
    out = pl.pallas_call(
        functools.partial(_mlp_kernel, nl, tuple(acts)),
        grid=(rp // tile,),
        in_specs=in_specs,
        out_specs=pl.BlockSpec((tile, pdims[-1]), lambda i: (i, 0)),
        out_shape=jax.ShapeDtypeStruct((rp, pdims[-1]), jnp.float32),
    )(*ops)
    return out[:rows, :dims[-1]]


def _fold_chain(layers, hidden_act, last_act='none'):
    ws, bs, acts = [], [], []
    nl = len(layers)
    inv = 1.0 / jnp.sqrt(1.0 + _EPS)
    for i, p in enumerate(layers):
        if 'g' in p:
            sc = inv * p['g']
            ws.append(p['W'] * sc[None, :])
            bs.append(p['b'] * sc + p['bt'])
        else:
            ws.append(p['W'])
            bs.append(p['b'])
        acts.append(hidden_act if i < nl - 1 else last_act)
    return ws, bs, acts


def _radius(pos_x, pos_y, r, max_n):
    d2 = (jnp.sum(pos_y * pos_y, -1)[:, None]
          + jnp.sum(pos_x * pos_x, -1)[None, :]
          - 2.0 * pos_y @ pos_x.T)
    n = pos_x.shape[0]
    score = jnp.where(d2 <= r * r, jnp.arange(n, dtype=jnp.int32)[None, :], n)
    neg, _ = jax.lax.top_k(-score, max_n)
    sc = -neg
    valid = sc < n
    return jnp.where(valid, sc, 0), valid


def _sa(xg, posg, r, m, ws, bs, acts):
    nb, n, dch = xg.shape
    idx = _fps_pallas(posg, m)
    posy = jnp.take_along_axis(posg, idx[..., None], axis=1)
    nbr, valid = jax.vmap(lambda px, py: _radius(px, py, r, 32))(posg, posy)
    nf = nbr.reshape(nb, m * 32, 1)
    fx = jnp.take_along_axis(xg, nf, axis=1).reshape(nb, m, 32, dch)
    fp = jnp.take_along_axis(posg, nf, axis=1).reshape(nb, m, 32, 3)
    fp = fp - posy[:, :, None, :]
    feat = jnp.concatenate([fx, fp], -1)
    msg = _mlp_pallas(feat.reshape(-1, dch + 3), ws, bs, acts)
    msg = msg.reshape(nb, m, 32, -1)
    msg = jnp.where(valid[..., None], msg, -jnp.inf)
    out = jnp.max(msg, axis=2)
    out = jnp.where(jnp.any(valid, axis=2)[..., None], out, 0.0)
    return jax.nn.relu(out), posy


def _up(xl, posl, xf, posf, cw, cb, ca, uw, ub, ua):
    nb, m, cl = xl.shape
    nf = posf.shape[1]
    d2 = (jnp.sum(posl * posl, -1)[:, :, None]
          + jnp.sum(posf * posf, -1)[:, None, :]
          - 2.0 * jnp.einsum('bmc,bnc->bmn', posl, posf))
    _, nbr = jax.lax.top_k(-d2, 64)
    pn = jnp.take_along_axis(posf, nbr.reshape(nb, m * 64, 1), axis=1)
    pn = pn.reshape(nb, m, 64, 3)
    feat = jnp.concatenate(
        [jnp.broadcast_to(xl[:, :, None, :], (nb, m, 64, cl)),
         posl[:, :, None, :] - pn], -1)
    feat_s = jnp.concatenate([xl, posl - posf[:, :m]], -1)
    allrows = jnp.concatenate(
        [feat.reshape(-1, cl + 3), feat_s.reshape(-1, cl + 3)], 0)
    allmsg = _mlp_pallas(allrows, cw, cb, ca)
    f = allmsg.shape[-1]
    msg = allmsg[:nb * m * 64].reshape(nb, m, 64, f)
    msg_s = allmsg[nb * m * 64:].reshape(nb, m, f)
    out0 = jnp.full((nf, f), -jnp.inf, jnp.float32)

    def scat(nbi, ms, mss):
        o = out0.at[nbi.reshape(-1)].max(ms.reshape(-1, f))
        return o.at[jnp.arange(m)].max(mss)

    out = jax.vmap(scat)(nbr, msg, msg_s)
    out = jnp.where(jnp.isneginf(out), 0.0, out)
    xc = jnp.concatenate([out, xf], -1)
    h = _mlp_pallas(xc.reshape(-1, xc.shape[-1]), uw, ub, ua)
    return h.reshape(nb, nf, -1)


def kernel(x, pos, batch, params):
    n_total = pos.shape[0]
    n = n_total // _B
    xg = x.reshape(_B, n, -1)
    posg = pos.reshape(_B, n, 3)
    m1, m2 = n // 4, n // 16

    sa0 = _fold_chain(params['sa'][0], 'sigmoid')
    sa1 = _fold_chain(params['sa'][1], 'sigmoid')
    dec0 = _fold_chain(params['dec'][0], 'sigmoid')
    dec1 = _fold_chain(params['dec'][1], 'sigmoid')
    up0 = _fold_chain([params['up'][0]], 'relu', last_act='relu')
    up1 = _fold_chain([params['up'][1]], 'relu', last_act='relu')
    fc = _fold_chain(params['fc'], 'relu', last_act='sigmoid')

    x1, pos1 = _sa(xg, posg, 1.0, m1, *sa0)
    x2, pos2 = _sa(x1, pos1, 2.0, m2, *sa1)
    u1 = _up(x2, pos2, x1, pos1, *dec0, *up0)
    u0 = _up(u1, pos1, xg, posg, *dec1, *up1)
    out = _mlp_pallas(u0.reshape(n_total, -1), *fc)
    return (out, pos, batch)


# speedup vs baseline: 2.0598x; 1.0457x over previous
import functools

import jax
import jax.numpy as jnp
from jax.experimental import pallas as pl

_B = 4
_EPS = 1e-5


def _round_up(v, m):
    return (v + m - 1) // m * m


def _fps_kernel(n, m, nb, p_ref, out_ref):
    npad = p_ref.shape[1]
    mpad = out_ref.shape[1]
    pt = p_ref[...]
    x0 = pt[0:nb, :]
    x1 = pt[nb:2 * nb, :]
    x2 = pt[2 * nb:3 * nb, :]
    col = jax.lax.broadcasted_iota(jnp.int32, (nb, npad), 1)
    mcol = jax.lax.broadcasted_iota(jnp.int32, (nb, mpad), 1)
    valid = col < n

    def body(i, carry):
        dists, last, idxs = carry
        sel = col == last
        p0 = jnp.sum(jnp.where(sel, x0, 0.0), axis=1, keepdims=True)
        p1 = jnp.sum(jnp.where(sel, x1, 0.0), axis=1, keepdims=True)
        p2 = jnp.sum(jnp.where(sel, x2, 0.0), axis=1, keepdims=True)
        d = (x0 - p0) ** 2 + (x1 - p1) ** 2 + (x2 - p2) ** 2
        dists = jnp.minimum(dists, jnp.where(valid, d, -jnp.inf))
        mx = jnp.max(dists, axis=1, keepdims=True)
        nxt = jnp.min(jnp.where(dists == mx, col, npad), axis=1,
                      keepdims=True).astype(jnp.int32)
        idxs = jnp.where(mcol == i + 1, nxt, idxs)
        return dists, nxt, idxs

    dists0 = jnp.where(valid, jnp.inf, -jnp.inf)
    last0 = jnp.zeros((nb, 1), jnp.int32)
    idxs0 = jnp.zeros((nb, mpad), jnp.int32)
    _, _, idxs = jax.lax.fori_loop(0, m - 1, body, (dists0, last0, idxs0))
    out_ref[...] = idxs


def _fps_pallas(pos_b, m):
    nb, n, _ = pos_b.shape
    npad = _round_up(n, 128)
    mpad = _round_up(m, 128)
    p = jnp.moveaxis(pos_b, 2, 0).reshape(3 * nb, n)
    p = jnp.pad(p, ((0, 0), (0, npad - n)))
    out = pl.pallas_call(
        functools.partial(_fps_kernel, n, m, nb),
        out_shape=jax.ShapeDtypeStruct((nb, mpad), jnp.int32),
    )(p)
    return out[:, :m]


def _mlp_kernel(nlayers, acts, x_ref, *refs):
    out_ref = refs[-1]
    h = x_ref[...]
    for i in range(nlayers):
        w = refs[2 * i][...]
        b = refs[2 * i + 1][...]
        h = jnp.dot(h, w, preferred_element_type=jnp.float32) + b
        a = acts[i]
        if a == 'sigmoid':
            h = jax.nn.sigmoid(h)
        elif a == 'relu':
            h = jax.nn.relu(h)
    out_ref[...] = h


def _mlp_pallas(x, ws, bs, acts, tile=512):
    rows, c0 = x.shape
    nl = len(ws)
    dims = [c0] + [w.shape[1] for w in ws]
    pdims = [_round_up(c, 128) for c in dims]
    rp = _round_up(rows, tile)
    xp = jnp.zeros((rp, pdims[0]), jnp.float32).at[:rows, :c0].set(x)
    ops = [xp]
    in_specs = [pl.BlockSpec((tile, pdims[0]), lambda i: (i, 0))]
    for li in range(nl):
        wp = jnp.zeros((pdims[li], pdims[li + 1]), jnp.float32)
        wp = wp.at[:dims[li], :dims[li + 1]].set(ws[li])
        bp = jnp.zeros((1, pdims[li + 1]), jnp.float32)
        bp = bp.at[:, :dims[li + 1]].set(bs[li])
        ops += [wp, bp]
        in_specs.append(pl.BlockSpec(wp.shape, lambda i: (0, 0)))
        in_specs.append(pl.BlockSpec(bp.shape, lambda i: (0, 0)))
    out = pl.pallas_call(
        functools.partial(_mlp_kernel, nl, tuple(acts)),
        grid=(rp // tile,),
        in_specs=in_specs,
        out_specs=pl.BlockSpec((tile, pdims[-1]), lambda i: (i, 0)),
        out_shape=jax.ShapeDtypeStruct((rp, pdims[-1]), jnp.float32),
    )(*ops)
    return out[:rows, :dims[-1]]


def _fold_chain(layers, hidden_act, last_act='none'):
    ws, bs, acts = [], [], []
    nl = len(layers)
    inv = 1.0 / jnp.sqrt(1.0 + _EPS)
    for i, p in enumerate(layers):
        if 'g' in p:
            sc = inv * p['g']
            ws.append(p['W'] * sc[None, :])
            bs.append(p['b'] * sc + p['bt'])
        else:
            ws.append(p['W'])
            bs.append(p['b'])
        acts.append(hidden_act if i < nl - 1 else last_act)
    return ws, bs, acts


def _radius(pos_x, pos_y, r, max_n):
    d2 = (jnp.sum(pos_y * pos_y, -1)[:, None]
          + jnp.sum(pos_x * pos_x, -1)[None, :]
          - 2.0 * pos_y @ pos_x.T)
    n = pos_x.shape[0]
    score = jnp.where(d2 <= r * r, jnp.arange(n, dtype=jnp.int32)[None, :], n)
    neg, _ = jax.lax.top_k(-score, max_n)
    sc = -neg
    valid = sc < n
    return jnp.where(valid, sc, 0), valid


def _sa(xg, posg, r, m, ws, bs, acts):
    nb, n, dch = xg.shape
    idx = _fps_pallas(posg, m)
    posy = jnp.take_along_axis(posg, idx[..., None], axis=1)
    nbr, valid = jax.vmap(lambda px, py: _radius(px, py, r, 32))(posg, posy)
    nf = nbr.reshape(nb, m * 32, 1)
    fx = jnp.take_along_axis(xg, nf, axis=1).reshape(nb, m, 32, dch)
    fp = jnp.take_along_axis(posg, nf, axis=1).reshape(nb, m, 32, 3)
    fp = fp - posy[:, :, None, :]
    feat = jnp.concatenate([fx, fp], -1)
    msg = _mlp_pallas(feat.reshape(-1, dch + 3), ws, bs, acts)
    msg = msg.reshape(nb, m, 32, -1)
    msg = jnp.where(valid[..., None], msg, -jnp.inf)
    out = jnp.max(msg, axis=2)
    out = jnp.where(jnp.any(valid, axis=2)[..., None], out, 0.0)
    return jax.nn.relu(out), posy


def _up(xl, posl, xf, posf, cw, cb, ca, uw, ub, ua):
    nb, m, cl = xl.shape
    nf = posf.shape[1]
    d2 = (jnp.sum(posl * posl, -1)[:, :, None]
          + jnp.sum(posf * posf, -1)[:, None, :]
          - 2.0 * jnp.einsum('bmc,bnc->bmn', posl, posf))
    _, nbr = jax.lax.top_k(-d2, 64)
    pn = jnp.take_along_axis(posf, nbr.reshape(nb, m * 64, 1), axis=1)
    pn = pn.reshape(nb, m, 64, 3)
    feat = jnp.concatenate(
        [jnp.broadcast_to(xl[:, :, None, :], (nb, m, 64, cl)),
         posl[:, :, None, :] - pn], -1)
    feat_s = jnp.concatenate([xl, posl - posf[:, :m]], -1)
    allrows = jnp.concatenate(
        [feat.reshape(-1, cl + 3), feat_s.reshape(-1, cl + 3)], 0)
    allmsg = _mlp_pallas(allrows, cw, cb, ca)
    f = allmsg.shape[-1]
    msg = allmsg[:nb * m * 64].reshape(nb, m, 64, f)
    msg_s = allmsg[nb * m * 64:].reshape(nb, m, f)
    out0 = jnp.full((nf, f), -jnp.inf, jnp.float32)

    def scat(nbi, ms, mss):
        o = out0.at[nbi.reshape(-1)].max(ms.reshape(-1, f))
        return o.at[jnp.arange(m)].max(mss)

    out = jax.vmap(scat)(nbr, msg, msg_s)
    out = jnp.where(jnp.isneginf(out), 0.0, out)
    xc = jnp.concatenate([out, xf], -1)
    h = _mlp_pallas(xc.reshape(-1, xc.shape[-1]), uw, ub, ua)
    return h.reshape(nb, nf, -1)


def kernel(x, pos, batch, params):
    n_total = pos.shape[0]
    n = n_total // _B
    xg = x.reshape(_B, n, -1)
    posg = pos.reshape(_B, n, 3)
    m1, m2 = n // 4, n // 16

    sa0 = _fold_chain(params['sa'][0], 'sigmoid')
    sa1 = _fold_chain(params['sa'][1], 'sigmoid')
    dec0 = _fold_chain(params['dec'][0], 'sigmoid')
    dec1 = _fold_chain(params['dec'][1], 'sigmoid')
    up0 = _fold_chain([params['up'][0]], 'relu', last_act='relu')
    up1 = _fold_chain([params['up'][1]], 'relu', last_act='relu')
    fc = _fold_chain(params['fc'], 'relu', last_act='sigmoid')

    x1, pos1 = _sa(xg, posg, 1.0, m1, *sa0)
    x2, pos2 = _sa(x1, pos1, 2.0, m2, *sa1)
    u1 = _up(x2, pos2, x1, pos1, *dec0, *up0)
    u0 = _up(u1, pos1, xg, posg, *dec1, *up1)
    out = _mlp_pallas(u0.reshape(n_total, -1), *fc)
    return (out, pos, batch)


# speedup vs baseline: 2.1065x; 1.0227x over previous
import functools

import jax
import jax.numpy as jnp
from jax.experimental import pallas as pl

_B = 4
_EPS = 1e-5


def _round_up(v, m):
    return (v + m - 1) // m * m


def _fps_kernel(n, m, nb, p_ref, out_ref):
    npad = p_ref.shape[1]
    mpad = out_ref.shape[1]
    pt = p_ref[...]
    x0 = pt[0:nb, :]
    x1 = pt[nb:2 * nb, :]
    x2 = pt[2 * nb:3 * nb, :]
    col = jax.lax.broadcasted_iota(jnp.int32, (nb, npad), 1)
    mcol = jax.lax.broadcasted_iota(jnp.int32, (nb, mpad), 1)
    valid = col < n

    def body(i, carry):
        dists, last, idxs = carry
        sel = col == last
        p0 = jnp.sum(jnp.where(sel, x0, 0.0), axis=1, keepdims=True)
        p1 = jnp.sum(jnp.where(sel, x1, 0.0), axis=1, keepdims=True)
        p2 = jnp.sum(jnp.where(sel, x2, 0.0), axis=1, keepdims=True)
        d = (x0 - p0) ** 2 + (x1 - p1) ** 2 + (x2 - p2) ** 2
        dists = jnp.minimum(dists, jnp.where(valid, d, -jnp.inf))
        mx = jnp.max(dists, axis=1, keepdims=True)
        nxt = jnp.min(jnp.where(dists == mx, col, npad), axis=1,
                      keepdims=True).astype(jnp.int32)
        idxs = jnp.where(mcol == i + 1, nxt, idxs)
        return dists, nxt, idxs

    dists0 = jnp.where(valid, jnp.inf, -jnp.inf)
    last0 = jnp.zeros((nb, 1), jnp.int32)
    idxs0 = jnp.zeros((nb, mpad), jnp.int32)
    _, _, idxs = jax.lax.fori_loop(0, m - 1, body, (dists0, last0, idxs0))
    out_ref[...] = idxs


def _fps_pallas(pos_b, m):
    nb, n, _ = pos_b.shape
    npad = _round_up(n, 128)
    mpad = _round_up(m, 128)
    p = jnp.moveaxis(pos_b, 2, 0).reshape(3 * nb, n)
    p = jnp.pad(p, ((0, 0), (0, npad - n)))
    out = pl.pallas_call(
        functools.partial(_fps_kernel, n, m, nb),
        out_shape=jax.ShapeDtypeStruct((nb, mpad), jnp.int32),
    )(p)
    return out[:, :m]


def _mlp_kernel(nlayers, acts, x_ref, *refs):
    out_ref = refs[-1]
    h = x_ref[...]
    for i in range(nlayers):
        w = refs[2 * i][...]
        b = refs[2 * i + 1][...]
        h = jnp.dot(h, w, preferred_element_type=jnp.float32) + b
        a = acts[i]
        if a == 'sigmoid':
            h = jax.nn.sigmoid(h)
        elif a == 'relu':
            h = jax.nn.relu(h)
    out_ref[...] = h


def _mlp_pallas(x, ws, bs, acts, tile=1024):
    rows, c0 = x.shape
    nl = len(ws)
    dims = [c0] + [w.shape[1] for w in ws]
    pdims = [_round_up(c, 128) for c in dims]
    rp = _round_up(rows, tile)
    xp = jnp.zeros((rp, pdims[0]), jnp.float32).at[:rows, :c0].set(x)
    ops = [xp]
    in_specs = [pl.BlockSpec((tile, pdims[0]), lambda i: (i, 0))]
    for li in range(nl):
        wp = jnp.zeros((pdims[li], pdims[li + 1]), jnp.float32)
        wp = wp.at[:dims[li], :dims[li + 1]].set(ws[li])
        bp = jnp.zeros((1, pdims[li + 1]), jnp.float32)
        bp = bp.at[:, :dims[li + 1]].set(bs[li])
        ops += [wp, bp]
        in_specs.append(pl.BlockSpec(wp.shape, lambda i: (0, 0)))
        in_specs.append(pl.BlockSpec(bp.shape, lambda i: (0, 0)))
    out = pl.pallas_call(
        functools.partial(_mlp_kernel, nl, tuple(acts)),
        grid=(rp // tile,),
        in_specs=in_specs,
        out_specs=pl.BlockSpec((tile, pdims[-1]), lambda i: (i, 0)),
        out_shape=jax.ShapeDtypeStruct((rp, pdims[-1]), jnp.float32),
    )(*ops)
    return out[:rows, :dims[-1]]


def _fold_chain(layers, hidden_act, last_act='none'):
    ws, bs, acts = [], [], []
    nl = len(layers)
    inv = 1.0 / jnp.sqrt(1.0 + _EPS)
    for i, p in enumerate(layers):
        if 'g' in p:
            sc = inv * p['g']
            ws.append(p['W'] * sc[None, :])
            bs.append(p['b'] * sc + p['bt'])
        else:
            ws.append(p['W'])
            bs.append(p['b'])
        acts.append(hidden_act if i < nl - 1 else last_act)
    return ws, bs, acts


def _radius(pos_x, pos_y, r, max_n):
    d2 = (jnp.sum(pos_y * pos_y, -1)[:, None]
          + jnp.sum(pos_x * pos_x, -1)[None, :]
          - 2.0 * pos_y @ pos_x.T)
    n = pos_x.shape[0]
    score = jnp.where(d2 <= r * r, jnp.arange(n, dtype=jnp.int32)[None, :], n)
    neg, _ = jax.lax.top_k(-score, max_n)
    sc = -neg
    valid = sc < n
    return jnp.where(valid, sc, 0), valid


def _sa(xg, posg, r, m, ws, bs, acts):
    nb, n, dch = xg.shape
    idx = _fps_pallas(posg, m)
    posy = jnp.take_along_axis(posg, idx[..., None], axis=1)
    nbr, valid = jax.vmap(lambda px, py: _radius(px, py, r, 32))(posg, posy)
    nf = nbr.reshape(nb, m * 32, 1)
    fx = jnp.take_along_axis(xg, nf, axis=1).reshape(nb, m, 32, dch)
    fp = jnp.take_along_axis(posg, nf, axis=1).reshape(nb, m, 32, 3)
    fp = fp - posy[:, :, None, :]
    feat = jnp.concatenate([fx, fp], -1)
    msg = _mlp_pallas(feat.reshape(-1, dch + 3), ws, bs, acts)
    msg = msg.reshape(nb, m, 32, -1)
    msg = jnp.where(valid[..., None], msg, -jnp.inf)
    out = jnp.max(msg, axis=2)
    out = jnp.where(jnp.any(valid, axis=2)[..., None], out, 0.0)
    return jax.nn.relu(out), posy


def _up(xl, posl, xf, posf, cw, cb, ca, uw, ub, ua):
    nb, m, cl = xl.shape
    nf = posf.shape[1]
    d2 = (jnp.sum(posl * posl, -1)[:, :, None]
          + jnp.sum(posf * posf, -1)[:, None, :]
          - 2.0 * jnp.einsum('bmc,bnc->bmn', posl, posf))
    _, nbr = jax.lax.top_k(-d2, 64)
    pn = jnp.take_along_axis(posf, nbr.reshape(nb, m * 64, 1), axis=1)
    pn = pn.reshape(nb, m, 64, 3)
    feat = jnp.concatenate(
        [jnp.broadcast_to(xl[:, :, None, :], (nb, m, 64, cl)),
         posl[:, :, None, :] - pn], -1)
    feat_s = jnp.concatenate([xl, posl - posf[:, :m]], -1)
    allrows = jnp.concatenate(
        [feat.reshape(-1, cl + 3), feat_s.reshape(-1, cl + 3)], 0)
    allmsg = _mlp_pallas(allrows, cw, cb, ca)
    f = allmsg.shape[-1]
    msg = allmsg[:nb * m * 64].reshape(nb, m, 64, f)
    msg_s = allmsg[nb * m * 64:].reshape(nb, m, f)
    out0 = jnp.full((nf, f), -jnp.inf, jnp.float32)

    def scat(nbi, ms, mss):
        o = out0.at[nbi.reshape(-1)].max(ms.reshape(-1, f))
        return o.at[jnp.arange(m)].max(mss)

    out = jax.vmap(scat)(nbr, msg, msg_s)
    out = jnp.where(jnp.isneginf(out), 0.0, out)
    xc = jnp.concatenate([out, xf], -1)
    h = _mlp_pallas(xc.reshape(-1, xc.shape[-1]), uw, ub, ua)
    return h.reshape(nb, nf, -1)


def kernel(x, pos, batch, params):
    n_total = pos.shape[0]
    n = n_total // _B
    xg = x.reshape(_B, n, -1)
    posg = pos.reshape(_B, n, 3)
    m1, m2 = n // 4, n // 16

    sa0 = _fold_chain(params['sa'][0], 'sigmoid')
    sa1 = _fold_chain(params['sa'][1], 'sigmoid')
    dec0 = _fold_chain(params['dec'][0], 'sigmoid')
    dec1 = _fold_chain(params['dec'][1], 'sigmoid')
    up0 = _fold_chain([params['up'][0]], 'relu', last_act='relu')
    up1 = _fold_chain([params['up'][1]], 'relu', last_act='relu')
    fc = _fold_chain(params['fc'], 'relu', last_act='sigmoid')

    x1, pos1 = _sa(xg, posg, 1.0, m1, *sa0)
    x2, pos2 = _sa(x1, pos1, 2.0, m2, *sa1)
    u1 = _up(x2, pos2, x1, pos1, *dec0, *up0)
    u0 = _up(u1, pos1, xg, posg, *dec1, *up1)
    out = _mlp_pallas(u0.reshape(n_total, -1), *fc)
    return (out, pos, batch)
